# baseline (speedup 1.0000x reference)
import functools

import jax
import jax.numpy as jnp
from jax import lax
from jax.experimental import pallas as pl
from jax.experimental.pallas import tpu as pltpu
from jax.experimental.pallas import tpu_sc as plsc

N_NODES = 10000
D = 128
N_EDGES = 320000
LN_EPS = 1e-5

NC = 2
NS = 16
NW = NC * NS
CHUNK = 64
NCH = 160
E_PAD = NW * NCH * CHUNK
ACCN = N_NODES + 16
RB = 2
ZR = 624
TAIL = N_NODES - NS * ZR


def _aggregate(x, src3, dst3, zrows):
    mesh = plsc.VectorSubcoreMesh(core_axis_name="c", subcore_axis_name="s")

    @functools.partial(
        pl.kernel,
        out_type=jax.ShapeDtypeStruct((NC * N_NODES, D), jnp.float32),
        mesh=mesh,
        scratch_types=[
            pltpu.VMEM((NCH, CHUNK), jnp.int32),
            pltpu.VMEM((NCH, CHUNK), jnp.int32),
            [pltpu.VMEM((CHUNK, D), jnp.float32)] * RB,
            pltpu.VMEM_SHARED((ACCN, D), jnp.float32),
            [pltpu.SemaphoreType.DMA] * RB,
        ],
    )
    def k(x_hbm, src_hbm, dst_hbm, z_hbm, out_hbm, src_v, dst_v, rows, acc,
          rowsem):
        c = lax.axis_index("c")
        s = lax.axis_index("s")
        wid = c * NS + s
        pltpu.sync_copy(z_hbm, acc.at[pl.ds(s * ZR, ZR)])

        @pl.when(s == 0)
        def _zero_tail():
            pltpu.sync_copy(z_hbm.at[pl.ds(0, TAIL)], acc.at[pl.ds(NS * ZR, TAIL)])
        pltpu.sync_copy(src_hbm.at[wid], src_v)
        pltpu.sync_copy(dst_hbm.at[wid], dst_v)
        plsc.subcore_barrier()
        def body(t, carry):
            pltpu.async_copy(x_hbm.at[src_v.at[t]], rows[0], rowsem[0]).wait()
            pltpu.sync_copy(rows[0], acc.at[dst_v.at[t]], add=True)
            return carry

        lax.fori_loop(0, NCH, body, 0)
        plsc.subcore_barrier()
        pltpu.sync_copy(
            acc.at[pl.ds(s * ZR, ZR)],
            out_hbm.at[pl.ds(c * N_NODES + s * ZR, ZR)],
        )

        @pl.when(s == 0)
        def _write_tail():
            pltpu.sync_copy(
                acc.at[pl.ds(NS * ZR, TAIL)],
                out_hbm.at[pl.ds(c * N_NODES + NS * ZR, TAIL)],
            )

    return k(x, src3, dst3, zrows)


def _mlp(x, p0, p1, W1, b1, g, bt, W2, b2, eps11):
    BR = 1000
    grid = (N_NODES // BR,)

    def body(eps_ref, x_ref, p0_ref, p1_ref, W1_ref, b1_ref, g_ref, bt_ref,
             W2_ref, b2_ref, o_ref):
        h = x_ref[...] * (1.0 + eps_ref[0, 0]) + p0_ref[...] + p1_ref[...]
        t = jnp.dot(h, W1_ref[...], preferred_element_type=jnp.float32) + b1_ref[...]
        mu = jnp.mean(t, axis=1, keepdims=True)
        d = t - mu
        var = jnp.mean(d * d, axis=1, keepdims=True)
        t = d * lax.rsqrt(var + LN_EPS) * g_ref[...] + bt_ref[...]
        t = jnp.maximum(t, 0.0)
        o_ref[...] = jnp.dot(t, W2_ref[...], preferred_element_type=jnp.float32) + b2_ref[...]

    row = lambda i: (i, 0)
    fixed = lambda i: (0, 0)
    return pl.pallas_call(
        body,
        grid=grid,
        in_specs=[
            pl.BlockSpec(memory_space=pltpu.MemorySpace.SMEM),
            pl.BlockSpec((BR, D), row),
            pl.BlockSpec((BR, D), row),
            pl.BlockSpec((BR, D), row),
            pl.BlockSpec((D, D), fixed),
            pl.BlockSpec((1, D), fixed),
            pl.BlockSpec((1, D), fixed),
            pl.BlockSpec((1, D), fixed),
            pl.BlockSpec((D, D), fixed),
            pl.BlockSpec((1, D), fixed),
        ],
        out_specs=pl.BlockSpec((BR, D), row),
        out_shape=jax.ShapeDtypeStruct((N_NODES, D), jnp.float32),
    )(eps11, x, p0, p1, W1, b1, g, bt, W2, b2)


def kernel(x, edge_index, W1, b1, ln_gamma, ln_beta, W2, b2, eps):
    ei = edge_index.astype(jnp.int32)
    npad = E_PAD - N_EDGES
    src3 = jnp.concatenate([ei[0], jnp.zeros((npad,), jnp.int32)]).reshape(NW, NCH, CHUNK)
    dst3 = jnp.concatenate([ei[1], jnp.full((npad,), N_NODES, jnp.int32)]).reshape(NW, NCH, CHUNK)
    zrows = jnp.zeros((ZR, D), jnp.float32)
    parts = _aggregate(x, src3, dst3, zrows)
    p0 = parts[:N_NODES]
    p1 = parts[N_NODES:]
    return _mlp(
        x, p0, p1, W1,
        b1.reshape(1, D), ln_gamma.reshape(1, D), ln_beta.reshape(1, D),
        W2, b2.reshape(1, D), eps.reshape(1, 1),
    )


# speedup vs baseline: 2.9486x; 2.9486x over previous
import functools

import jax
import jax.numpy as jnp
from jax import lax
from jax.experimental import pallas as pl
from jax.experimental.pallas import tpu as pltpu
from jax.experimental.pallas import tpu_sc as plsc

N_NODES = 10000
D = 128
N_EDGES = 320000
LN_EPS = 1e-5

NC = 2
NS = 16
NW = NC * NS
CHUNK = 80
NCH = 125
EPW = NCH * CHUNK
RB = 2
ZR = 624
TAIL = N_NODES - NS * ZR


def _aggregate(x, src3, dst3, zrows):
    mesh = plsc.VectorSubcoreMesh(core_axis_name="c", subcore_axis_name="s")

    @functools.partial(
        pl.kernel,
        out_type=jax.ShapeDtypeStruct((NC * N_NODES, D), jnp.float32),
        mesh=mesh,
        scratch_types=[
            pltpu.VMEM((EPW,), jnp.int32),
            pltpu.VMEM((NCH, CHUNK), jnp.int32),
            [pltpu.VMEM((CHUNK, D), jnp.float32)] * RB,
            pltpu.VMEM_SHARED((N_NODES, D), jnp.float32),
            [pltpu.SemaphoreType.DMA] * RB,
        ],
    )
    def k(x_hbm, src_hbm, dst_hbm, z_hbm, out_hbm, src_v, dst_v, rows, acc,
          rowsem):
        c = lax.axis_index("c")
        s = lax.axis_index("s")
        wid = c * NS + s
        pltpu.sync_copy(z_hbm, acc.at[pl.ds(s * ZR, ZR)])

        @pl.when(s == 0)
        def _zero_tail():
            pltpu.sync_copy(z_hbm.at[pl.ds(0, TAIL)], acc.at[pl.ds(NS * ZR, TAIL)])
        pltpu.sync_copy(src_hbm.at[wid], src_v)
        pltpu.sync_copy(dst_hbm.at[wid], dst_v)
        plsc.subcore_barrier()
        pltpu.async_copy(x_hbm.at[src_v.at[pl.ds(0, CHUNK)]], rows[0], rowsem[0])

        def gather_wait(t, b):
            pltpu.make_async_copy(
                x_hbm.at[src_v.at[pl.ds(t * CHUNK, CHUNK)]], rows[b], rowsem[b]).wait()

        def gather_start(t, b):
            pltpu.async_copy(
                x_hbm.at[src_v.at[pl.ds(t * CHUNK, CHUNK)]], rows[b], rowsem[b])

        def body(g, carry):
            for u in range(RB):
                t = g * RB + u
                b = u
                gather_wait(t, b)

                @pl.when(t + 1 < NCH)
                def _issue_next():
                    gather_start(t + 1, (u + 1) % RB)

                pltpu.sync_copy(rows[b], acc.at[dst_v.at[t]], add=True)
            return carry

        lax.fori_loop(0, (NCH - 1) // RB, body, 0)
        gather_wait(NCH - 1, (NCH - 1) % RB)
        pltpu.sync_copy(rows[(NCH - 1) % RB], acc.at[dst_v.at[NCH - 1]], add=True)
        plsc.subcore_barrier()
        pltpu.sync_copy(
            acc.at[pl.ds(s * ZR, ZR)],
            out_hbm.at[pl.ds(c * N_NODES + s * ZR, ZR)],
        )

        @pl.when(s == 0)
        def _write_tail():
            pltpu.sync_copy(
                acc.at[pl.ds(NS * ZR, TAIL)],
                out_hbm.at[pl.ds(c * N_NODES + NS * ZR, TAIL)],
            )

    return k(x, src3, dst3, zrows)


def _mlp(x, p0, p1, W1, b1, g, bt, W2, b2, eps11):
    BR = 1000
    grid = (N_NODES // BR,)

    def body(eps_ref, x_ref, p0_ref, p1_ref, W1_ref, b1_ref, g_ref, bt_ref,
             W2_ref, b2_ref, o_ref):
        h = x_ref[...] * (1.0 + eps_ref[0, 0]) + p0_ref[...] + p1_ref[...]
        t = jnp.dot(h, W1_ref[...], preferred_element_type=jnp.float32) + b1_ref[...]
        mu = jnp.mean(t, axis=1, keepdims=True)
        d = t - mu
        var = jnp.mean(d * d, axis=1, keepdims=True)
        t = d * lax.rsqrt(var + LN_EPS) * g_ref[...] + bt_ref[...]
        t = jnp.maximum(t, 0.0)
        o_ref[...] = jnp.dot(t, W2_ref[...], preferred_element_type=jnp.float32) + b2_ref[...]

    row = lambda i: (i, 0)
    fixed = lambda i: (0, 0)
    return pl.pallas_call(
        body,
        grid=grid,
        in_specs=[
            pl.BlockSpec(memory_space=pltpu.MemorySpace.SMEM),
            pl.BlockSpec((BR, D), row),
            pl.BlockSpec((BR, D), row),
            pl.BlockSpec((BR, D), row),
            pl.BlockSpec((D, D), fixed),
            pl.BlockSpec((1, D), fixed),
            pl.BlockSpec((1, D), fixed),
            pl.BlockSpec((1, D), fixed),
            pl.BlockSpec((D, D), fixed),
            pl.BlockSpec((1, D), fixed),
        ],
        out_specs=pl.BlockSpec((BR, D), row),
        out_shape=jax.ShapeDtypeStruct((N_NODES, D), jnp.float32),
    )(eps11, x, p0, p1, W1, b1, g, bt, W2, b2)


def kernel(x, edge_index, W1, b1, ln_gamma, ln_beta, W2, b2, eps):
    ei = edge_index.astype(jnp.int32)
    src3 = ei[0].reshape(NW, EPW)
    dst3 = ei[1].reshape(NW, NCH, CHUNK)
    zrows = jnp.zeros((ZR, D), jnp.float32)
    parts = _aggregate(x, src3, dst3, zrows)
    p0 = parts[:N_NODES]
    p1 = parts[N_NODES:]
    return _mlp(
        x, p0, p1, W1,
        b1.reshape(1, D), ln_gamma.reshape(1, D), ln_beta.reshape(1, D),
        W2, b2.reshape(1, D), eps.reshape(1, 1),
    )


# speedup vs baseline: 2.9489x; 1.0001x over previous
import functools

import jax
import jax.numpy as jnp
from jax import lax
from jax.experimental import pallas as pl
from jax.experimental.pallas import tpu as pltpu
from jax.experimental.pallas import tpu_sc as plsc

N_NODES = 10000
D = 128
N_EDGES = 320000
LN_EPS = 1e-5

NC = 2
NS = 16
NW = NC * NS
CHUNK = 80
NCH = 125
EPW = NCH * CHUNK
RB = 2
ZR = 624
TAIL = N_NODES - NS * ZR


def _aggregate(x, src3, dst3, zrows):
    mesh = plsc.VectorSubcoreMesh(core_axis_name="c", subcore_axis_name="s")

    @functools.partial(
        pl.kernel,
        out_type=jax.ShapeDtypeStruct((NC * N_NODES, D), jnp.float32),
        mesh=mesh,
        scratch_types=[
            pltpu.VMEM((EPW,), jnp.int32),
            pltpu.VMEM((NCH, CHUNK), jnp.int32),
            [pltpu.VMEM((CHUNK, D), jnp.float32)] * RB,
            pltpu.VMEM_SHARED((N_NODES, D), jnp.float32),
            [pltpu.SemaphoreType.DMA] * RB,
            [pltpu.SemaphoreType.DMA] * RB,
        ],
    )
    def k(x_hbm, src_hbm, dst_hbm, z_hbm, out_hbm, src_v, dst_v, rows, acc,
          rowsem, scatsem):
        c = lax.axis_index("c")
        s = lax.axis_index("s")
        wid = c * NS + s
        pltpu.sync_copy(z_hbm, acc.at[pl.ds(s * ZR, ZR)])

        @pl.when(s == 0)
        def _zero_tail():
            pltpu.sync_copy(z_hbm.at[pl.ds(0, TAIL)], acc.at[pl.ds(NS * ZR, TAIL)])
        pltpu.sync_copy(src_hbm.at[wid], src_v)
        pltpu.sync_copy(dst_hbm.at[wid], dst_v)
        plsc.subcore_barrier()
        pltpu.async_copy(x_hbm.at[src_v.at[pl.ds(0, CHUNK)]], rows[0], rowsem[0])

        def gather_wait(t, b):
            pltpu.make_async_copy(
                x_hbm.at[src_v.at[pl.ds(t * CHUNK, CHUNK)]], rows[b], rowsem[b]).wait()

        def gather_start(t, b):
            pltpu.async_copy(
                x_hbm.at[src_v.at[pl.ds(t * CHUNK, CHUNK)]], rows[b], rowsem[b])

        def scat_wait(t, b):
            pltpu.make_async_copy(rows[b], acc.at[dst_v.at[t]], scatsem[b]).wait()

        def body(g, carry):
            for u in range(RB):
                t = g * RB + u
                b = u
                bn = (u + 1) % RB
                gather_wait(t, b)

                @pl.when(t > 0)
                def _wait_prev_scatter():
                    scat_wait(t - 1, bn)

                @pl.when(t + 1 < NCH)
                def _issue_next():
                    gather_start(t + 1, bn)

                pltpu.make_async_copy(
                    rows[b], acc.at[dst_v.at[t]], scatsem[b]).start(add=True)
            return carry

        lax.fori_loop(0, (NCH - 1) // RB, body, 0)
        tb = (NCH - 1) % RB
        gather_wait(NCH - 1, tb)
        scat_wait(NCH - 2, (tb + 1) % RB)
        pltpu.make_async_copy(
            rows[tb], acc.at[dst_v.at[NCH - 1]], scatsem[tb]).start(add=True)
        scat_wait(NCH - 1, tb)
        plsc.subcore_barrier()
        pltpu.sync_copy(
            acc.at[pl.ds(s * ZR, ZR)],
            out_hbm.at[pl.ds(c * N_NODES + s * ZR, ZR)],
        )

        @pl.when(s == 0)
        def _write_tail():
            pltpu.sync_copy(
                acc.at[pl.ds(NS * ZR, TAIL)],
                out_hbm.at[pl.ds(c * N_NODES + NS * ZR, TAIL)],
            )

    return k(x, src3, dst3, zrows)


def _mlp(x, p0, p1, W1, b1, g, bt, W2, b2, eps11):
    BR = 1000
    grid = (N_NODES // BR,)

    def body(eps_ref, x_ref, p0_ref, p1_ref, W1_ref, b1_ref, g_ref, bt_ref,
             W2_ref, b2_ref, o_ref):
        h = x_ref[...] * (1.0 + eps_ref[0, 0]) + p0_ref[...] + p1_ref[...]
        t = jnp.dot(h, W1_ref[...], preferred_element_type=jnp.float32) + b1_ref[...]
        mu = jnp.mean(t, axis=1, keepdims=True)
        d = t - mu
        var = jnp.mean(d * d, axis=1, keepdims=True)
        t = d * lax.rsqrt(var + LN_EPS) * g_ref[...] + bt_ref[...]
        t = jnp.maximum(t, 0.0)
        o_ref[...] = jnp.dot(t, W2_ref[...], preferred_element_type=jnp.float32) + b2_ref[...]

    row = lambda i: (i, 0)
    fixed = lambda i: (0, 0)
    return pl.pallas_call(
        body,
        grid=grid,
        in_specs=[
            pl.BlockSpec(memory_space=pltpu.MemorySpace.SMEM),
            pl.BlockSpec((BR, D), row),
            pl.BlockSpec((BR, D), row),
            pl.BlockSpec((BR, D), row),
            pl.BlockSpec((D, D), fixed),
            pl.BlockSpec((1, D), fixed),
            pl.BlockSpec((1, D), fixed),
            pl.BlockSpec((1, D), fixed),
            pl.BlockSpec((D, D), fixed),
            pl.BlockSpec((1, D), fixed),
        ],
        out_specs=pl.BlockSpec((BR, D), row),
        out_shape=jax.ShapeDtypeStruct((N_NODES, D), jnp.float32),
    )(eps11, x, p0, p1, W1, b1, g, bt, W2, b2)


def kernel(x, edge_index, W1, b1, ln_gamma, ln_beta, W2, b2, eps):
    ei = edge_index.astype(jnp.int32)
    src3 = ei[0].reshape(NW, EPW)
    dst3 = ei[1].reshape(NW, NCH, CHUNK)
    zrows = jnp.zeros((ZR, D), jnp.float32)
    parts = _aggregate(x, src3, dst3, zrows)
    p0 = parts[:N_NODES]
    p1 = parts[N_NODES:]
    return _mlp(
        x, p0, p1, W1,
        b1.reshape(1, D), ln_gamma.reshape(1, D), ln_beta.reshape(1, D),
        W2, b2.reshape(1, D), eps.reshape(1, 1),
    )


# speedup vs baseline: 4.6238x; 1.5680x over previous
import functools

import jax
import jax.numpy as jnp
from jax import lax
from jax.experimental import pallas as pl
from jax.experimental.pallas import tpu as pltpu
from jax.experimental.pallas import tpu_sc as plsc

N_NODES = 10000
D = 128
N_EDGES = 320000
LN_EPS = 1e-5

NC = 2
NS = 16
NW = NC * NS
CHUNK = 80
NCH = 125
EPW = NCH * CHUNK
RB = 4
IB = 8
ZR = 624
TAIL = N_NODES - NS * ZR


def _aggregate(x, eidx, zrows):
    mesh = plsc.VectorSubcoreMesh(core_axis_name="c", subcore_axis_name="s")

    @functools.partial(
        pl.kernel,
        out_type=jax.ShapeDtypeStruct((NC * N_NODES, D), jnp.float32),
        mesh=mesh,
        scratch_types=[
            pltpu.VMEM((IB, CHUNK), jnp.int32),
            pltpu.VMEM((IB, CHUNK), jnp.int32),
            [pltpu.VMEM((CHUNK, D), jnp.float32)] * RB,
            pltpu.VMEM_SHARED((N_NODES, D), jnp.float32),
            [pltpu.SemaphoreType.DMA] * IB,
            [pltpu.SemaphoreType.DMA] * IB,
            [pltpu.SemaphoreType.DMA] * RB,
            [pltpu.SemaphoreType.DMA] * RB,
        ],
    )
    def k(x_hbm, e_hbm, z_hbm, out_hbm, sring, dring, rows, acc,
          isem_s, isem_d, rowsem, scatsem):
        c = lax.axis_index("c")
        s = lax.axis_index("s")
        wid = c * NS + s
        soff = wid * NCH
        doff = NW * NCH + wid * NCH
        for j in range(IB - 1):
            pltpu.async_copy(e_hbm.at[soff + j], sring.at[j], isem_s[j])
            pltpu.async_copy(e_hbm.at[doff + j], dring.at[j], isem_d[j])
        pltpu.sync_copy(z_hbm, acc.at[pl.ds(s * ZR, ZR)])

        @pl.when(s == 0)
        def _zero_tail():
            pltpu.sync_copy(z_hbm.at[pl.ds(0, TAIL)], acc.at[pl.ds(NS * ZR, TAIL)])
        for j in range(RB - 1):
            pltpu.make_async_copy(e_hbm.at[soff + j], sring.at[j], isem_s[j]).wait()
            pltpu.async_copy(x_hbm.at[sring.at[j]], rows[j], rowsem[j])
        plsc.subcore_barrier()

        def step(t, u):
            b = u % RB
            q = u % IB
            bp = (u + RB - 1) % RB
            qp = (u + IB - 1) % IB
            qg = (u + RB - 1) % IB
            pltpu.make_async_copy(x_hbm.at[sring.at[q]], rows[b], rowsem[b]).wait()
            pltpu.make_async_copy(e_hbm.at[doff + t], dring.at[q], isem_d[q]).wait()
            pltpu.make_async_copy(rows[b], acc.at[dring.at[q]], scatsem[b]).start(add=True)

            @pl.when(t > 0)
            def _wait_prev_scatter():
                pltpu.make_async_copy(rows[bp], acc.at[dring.at[qp]], scatsem[bp]).wait()

            @pl.when(t + IB - 1 < NCH)
            def _issue_idx():
                pltpu.async_copy(e_hbm.at[soff + t + IB - 1], sring.at[qp], isem_s[qp])
                pltpu.async_copy(e_hbm.at[doff + t + IB - 1], dring.at[qp], isem_d[qp])

            @pl.when(t + RB - 1 < NCH)
            def _issue_gather():
                pltpu.make_async_copy(
                    e_hbm.at[soff + t + RB - 1], sring.at[qg], isem_s[qg]).wait()
                pltpu.async_copy(x_hbm.at[sring.at[qg]], rows[bp], rowsem[bp])

        def body(g, carry):
            for u in range(IB):
                step(g * IB + u, u)
            return carry

        lax.fori_loop(0, NCH // IB, body, 0)
        for u in range(NCH - (NCH // IB) * IB):
            step((NCH // IB) * IB + u, u)
        pltpu.make_async_copy(
            rows[(NCH - 1) % RB], acc.at[dring.at[(NCH - 1) % IB]],
            scatsem[(NCH - 1) % RB]).wait()
        plsc.subcore_barrier()
        pltpu.sync_copy(
            acc.at[pl.ds(s * ZR, ZR)],
            out_hbm.at[pl.ds(c * N_NODES + s * ZR, ZR)],
        )

        @pl.when(s == 0)
        def _write_tail():
            pltpu.sync_copy(
                acc.at[pl.ds(NS * ZR, TAIL)],
                out_hbm.at[pl.ds(c * N_NODES + NS * ZR, TAIL)],
            )

    return k(x, eidx, zrows)


def _mlp(x, parts, W1, b1, g, bt, W2, b2, eps11):
    BR = 2000
    grid = (N_NODES // BR,)

    def body(eps_ref, x_ref, p0_ref, p1_ref, W1_ref, b1_ref, g_ref, bt_ref,
             W2_ref, b2_ref, o_ref):
        h = x_ref[...] * (1.0 + eps_ref[0, 0]) + p0_ref[...] + p1_ref[...]
        t = jnp.dot(h, W1_ref[...], preferred_element_type=jnp.float32) + b1_ref[...]
        mu = jnp.mean(t, axis=1, keepdims=True)
        d = t - mu
        var = jnp.mean(d * d, axis=1, keepdims=True)
        t = d * lax.rsqrt(var + LN_EPS) * g_ref[...] + bt_ref[...]
        t = jnp.maximum(t, 0.0)
        o_ref[...] = jnp.dot(t, W2_ref[...], preferred_element_type=jnp.float32) + b2_ref[...]

    row = lambda i: (i, 0)
    row1 = lambda i: (N_NODES // BR + i, 0)
    fixed = lambda i: (0, 0)
    return pl.pallas_call(
        body,
        grid=grid,
        in_specs=[
            pl.BlockSpec(memory_space=pltpu.MemorySpace.SMEM),
            pl.BlockSpec((BR, D), row),
            pl.BlockSpec((BR, D), row),
            pl.BlockSpec((BR, D), row1),
            pl.BlockSpec((D, D), fixed),
            pl.BlockSpec((1, D), fixed),
            pl.BlockSpec((1, D), fixed),
            pl.BlockSpec((1, D), fixed),
            pl.BlockSpec((D, D), fixed),
            pl.BlockSpec((1, D), fixed),
        ],
        out_specs=pl.BlockSpec((BR, D), row),
        out_shape=jax.ShapeDtypeStruct((N_NODES, D), jnp.float32),
    )(eps11, x, parts, parts, W1, b1, g, bt, W2, b2)


def kernel(x, edge_index, W1, b1, ln_gamma, ln_beta, W2, b2, eps):
    eidx = edge_index.astype(jnp.int32).reshape(2 * NW * NCH, CHUNK)
    zrows = jnp.zeros((ZR, D), jnp.float32)
    parts = _aggregate(x, eidx, zrows)
    return _mlp(
        x, parts, W1,
        b1.reshape(1, D), ln_gamma.reshape(1, D), ln_beta.reshape(1, D),
        W2, b2.reshape(1, D), eps.reshape(1, 1),
    )
